# baseline (speedup 1.0000x reference)
import functools

import jax
import jax.numpy as jnp
from jax import lax
from jax.experimental import pallas as pl
from jax.experimental.pallas import tpu as pltpu
from jax.experimental.pallas import tpu_sc as plsc

N = 10000
E = 320000
D_IN = 1310
H = 128
B = 512
NH = 4

NC = 2
NS = 16
LANES = 16
NW = NC * NS
K = 80
CHUNKS_D = E // (NW * K)
CHUNKS_A = E // (NS * K)
NP = 10240
RPT = NP // NS


def _wid():
    return lax.axis_index("s") * NC + lax.axis_index("c")


def _sc_deg_body(colr, wr, out, colbuf, wbuf, zb, degsh, sem):
    cid = lax.axis_index("c")
    sid = lax.axis_index("s")
    wid = _wid()

    pltpu.sync_copy(colr.at[wid], colbuf)
    pltpu.sync_copy(wr.at[wid], wbuf)

    zeros16 = jnp.zeros((LANES,), jnp.float32)

    @pl.loop(0, RPT // LANES)
    def _(j):
        zb[pl.ds(j * LANES, LANES)] = zeros16

    pltpu.sync_copy(zb, degsh.at[pl.ds(sid * RPT, RPT)])

    plsc.subcore_barrier()

    for base in range(0, CHUNKS_D, 25):
        for k in range(base, base + 25):
            pltpu.make_async_copy(
                wbuf.at[k], degsh.at[colbuf.at[k]], sem
            ).start(add=True)
        for k in range(base, base + 25):
            pltpu.make_async_copy(wbuf.at[k], degsh.at[colbuf.at[k]], sem).wait()

    plsc.subcore_barrier()

    pltpu.sync_copy(
        degsh.at[pl.ds(sid * RPT, RPT)],
        out.at[cid].at[pl.ds(sid * RPT, RPT)],
    )


def _sc_agg_body(g, rowr, colr, wr, out, rb0, rb1, cb0, cb1, wb0, wb1,
                 G0, G1, Sbuf, aggsh, semg0, semg1, semi0, semi1):
    sid = lax.axis_index("s")

    zeros16 = jnp.zeros((LANES,), jnp.float32)
    lanes = lax.iota(jnp.int32, LANES)

    @pl.loop(0, K)
    def _(r):
        for c in range(H // LANES):
            G0[r, pl.ds(c * LANES, LANES)] = zeros16

    for rep in range(RPT // K):
        pltpu.sync_copy(G0, aggsh.at[pl.ds(sid * RPT + rep * K, K)])

    plsc.subcore_barrier()

    src_row = rowr.at[sid]
    src_col = colr.at[sid]
    src_w = wr.at[sid]

    def idx_start(k, rb, cb, wb, semi):
        pltpu.make_async_copy(src_row.at[k], rb, semi).start()
        pltpu.make_async_copy(src_col.at[k], cb, semi).start()
        pltpu.make_async_copy(src_w.at[k], wb, semi).start()

    def idx_wait(k, rb, cb, wb, semi):
        pltpu.make_async_copy(src_row.at[k], rb, semi).wait()
        pltpu.make_async_copy(src_col.at[k], cb, semi).wait()
        pltpu.make_async_copy(src_w.at[k], wb, semi).wait()

    def gather_start(rb, Gbuf, semg):
        pltpu.make_async_copy(g.at[rb.at[0]], Gbuf, semg).start()

    def gather_wait(rb, Gbuf, semg):
        pltpu.make_async_copy(g.at[rb.at[0]], Gbuf, semg).wait()

    zero16i = jnp.zeros((LANES,), jnp.int32)

    def scale_and_scatter(cb, wb, Gbuf):
        @plsc.parallel_loop(0, K, unroll=2)
        def _(e):
            e16 = jnp.full((LANES,), e, jnp.int32)
            we = plsc.load_gather(wb, [zero16i, e16])
            for c in range(H // LANES):
                Sbuf[e, pl.ds(c * LANES, LANES)] = (
                    Gbuf[e, pl.ds(c * LANES, LANES)] * we
                )

        pltpu.sync_copy(Sbuf, aggsh.at[cb.at[0]], add=True)

    bufs = ((rb0, cb0, wb0, G0, semg0, semi0),
            (rb1, cb1, wb1, G1, semg1, semi1))

    idx_start(0, rb0, cb0, wb0, semi0)
    idx_wait(0, rb0, cb0, wb0, semi0)
    gather_start(rb0, G0, semg0)
    idx_start(1, rb1, cb1, wb1, semi1)

    @pl.loop(0, CHUNKS_A, step=2)
    def _(k):
        for half in range(2):
            kk = k + half
            rb, cb, wb, Gbuf, semg, semi = bufs[half]
            rbn, cbn, wbn, Gn, semgn, semin = bufs[1 - half]

            @pl.when(kk + 1 < CHUNKS_A)
            def _():
                idx_wait(kk + 1, rbn, cbn, wbn, semin)
                gather_start(rbn, Gn, semgn)

            gather_wait(rb, Gbuf, semg)
            scale_and_scatter(cb, wb, Gbuf)

            @pl.when(kk + 2 < CHUNKS_A)
            def _():
                idx_start(kk + 2, rb, cb, wb, semi)

    plsc.subcore_barrier()

    pltpu.sync_copy(
        aggsh.at[pl.ds(sid * RPT, RPT)],
        out.at[pl.ds(sid * RPT, RPT)],
    )


@functools.cache
def _sc_mesh(num_cores):
    return plsc.VectorSubcoreMesh(
        core_axis_name="c", subcore_axis_name="s", num_cores=num_cores,
        num_subcores=NS,
    )


@functools.cache
def _sc_deg_kernel():
    return pl.kernel(
        _sc_deg_body,
        out_type=jax.ShapeDtypeStruct((NC, NP), jnp.float32),
        mesh=_sc_mesh(NC),
        scratch_types=[
            pltpu.VMEM((CHUNKS_D, K), jnp.int32),
            pltpu.VMEM((CHUNKS_D, K), jnp.float32),
            pltpu.VMEM((RPT,), jnp.float32),
            pltpu.VMEM_SHARED((NP,), jnp.float32),
            pltpu.SemaphoreType.DMA,
        ],
        compiler_params=pltpu.CompilerParams(needs_layout_passes=False),
    )


def _sc_deg(colr, wr):
    return _sc_deg_kernel()(colr, wr)


@functools.cache
def _sc_agg_kernel():
    return pl.kernel(
        _sc_agg_body,
        out_type=jax.ShapeDtypeStruct((NP, H), jnp.float32),
        mesh=_sc_mesh(1),
        scratch_types=[
            pltpu.VMEM((1, K), jnp.int32),
            pltpu.VMEM((1, K), jnp.int32),
            pltpu.VMEM((1, K), jnp.int32),
            pltpu.VMEM((1, K), jnp.int32),
            pltpu.VMEM((1, K), jnp.float32),
            pltpu.VMEM((1, K), jnp.float32),
            pltpu.VMEM((K, H), jnp.float32),
            pltpu.VMEM((K, H), jnp.float32),
            pltpu.VMEM((K, H), jnp.float32),
            pltpu.VMEM_SHARED((NP, H), jnp.float32),
            pltpu.SemaphoreType.DMA,
            pltpu.SemaphoreType.DMA,
            pltpu.SemaphoreType.DMA,
            pltpu.SemaphoreType.DMA,
        ],
        compiler_params=pltpu.CompilerParams(needs_layout_passes=False),
    )


def _sc_agg(g, rowr, colr, wr):
    return _sc_agg_kernel()(g, rowr, colr, wr)


_GRID = 10
_BLK = N // _GRID


def _ln(x, gamma, beta):
    mu = jnp.mean(x, axis=-1, keepdims=True)
    d = x - mu
    var = jnp.mean(d * d, axis=-1, keepdims=True)
    return d * lax.rsqrt(var + 1e-5) * gamma + beta


def _row_spec(cols):
    return pl.BlockSpec((_BLK, cols), lambda i: (i, 0))


def _full_spec(r, c):
    return pl.BlockSpec((r, c), lambda i: (0, 0))


def _tc_proj(x, proj_W, proj_b, proj_g, proj_bb):
    def body(x_ref, w_ref, b_ref, g_ref, bb_ref, o_ref):
        h = jnp.dot(x_ref[...], w_ref[...], preferred_element_type=jnp.float32)
        h = h + b_ref[...]
        o_ref[...] = jnp.maximum(_ln(h, g_ref[...], bb_ref[...]), 0.0)

    return pl.pallas_call(
        body,
        grid=(_GRID,),
        in_specs=[
            _row_spec(D_IN),
            _full_spec(D_IN, H),
            _full_spec(1, H),
            _full_spec(1, H),
            _full_spec(1, H),
        ],
        out_specs=_row_spec(H),
        out_shape=jax.ShapeDtypeStruct((N, H), jnp.float32),
    )(x, proj_W, proj_b, proj_g, proj_bb)


def _tc_gdis(h, W, deg0, deg1):
    def body(h_ref, w_ref, d0_ref, d1_ref, g_ref, dis_ref):
        deg = d0_ref[...] + d1_ref[...] + 1.0
        dis = lax.rsqrt(deg)
        dis_ref[...] = dis
        g_ref[...] = jnp.dot(
            h_ref[...], w_ref[...], preferred_element_type=jnp.float32
        ) * dis

    return pl.pallas_call(
        body,
        grid=(_GRID,),
        in_specs=[
            _row_spec(H),
            _full_spec(H, H),
            _row_spec(1),
            _row_spec(1),
        ],
        out_specs=[_row_spec(H), _row_spec(1)],
        out_shape=[
            jax.ShapeDtypeStruct((N, H), jnp.float32),
            jax.ShapeDtypeStruct((N, 1), jnp.float32),
        ],
    )(h, W, deg0, deg1)


def _tc_layer(a, g, h_prev, dis, bias, ln_g, ln_b, W_next, residual):
    def body(a_ref, g_ref, hp_ref, dis_ref, b_ref, lg_ref, lb_ref,
             w_ref, h_ref, gn_ref):
        dis = dis_ref[...]
        xn = dis * (a_ref[...] + g_ref[...]) + b_ref[...]
        if residual:
            xn = xn + hp_ref[...]
        h = jnp.maximum(_ln(xn, lg_ref[...], lb_ref[...]), 0.0)
        h_ref[...] = h
        gn_ref[...] = jnp.dot(
            h, w_ref[...], preferred_element_type=jnp.float32
        ) * dis

    return pl.pallas_call(
        body,
        grid=(_GRID,),
        in_specs=[
            _row_spec(H),
            _row_spec(H),
            _row_spec(H),
            _row_spec(1),
            _full_spec(1, H),
            _full_spec(1, H),
            _full_spec(1, H),
            _full_spec(H, H),
        ],
        out_specs=[_row_spec(H), _row_spec(H)],
        out_shape=[
            jax.ShapeDtypeStruct((N, H), jnp.float32),
            jax.ShapeDtypeStruct((N, H), jnp.float32),
        ],
    )(a, g, h_prev, dis, bias, ln_g, ln_b, W_next)


_ACC_C = 768


def _tc_final(a, g, h_prev, dis, bias, ln_g, ln_b, W1, b1, W2, b2,
              batch2d, c1W, c1b, c2W, c2b, c3W, c3b):
    def body(a_ref, g_ref, hp_ref, dis_ref, b_ref, lg_ref, lb_ref,
             w1_ref, b1_ref, w2_ref, b2_ref, batch_ref,
             c1w_ref, c1b_ref, c2w_ref, c2b_ref, c3w_ref, c3b_ref,
             o_ref, acc_ref):
        i = pl.program_id(0)
        xn = dis_ref[...] * (a_ref[...] + g_ref[...]) + b_ref[...]
        xn = xn + hp_ref[...]
        M = jnp.maximum(_ln(xn, lg_ref[...], lb_ref[...]), 0.0)

        t = jnp.tanh(
            jnp.dot(M, w1_ref[...], preferred_element_type=jnp.float32)
            + b1_ref[...]
        )
        scores = jnp.dot(t, w2_ref[...], preferred_element_type=jnp.float32) \
            + b2_ref[...]
        Eb = jnp.exp(scores)

        gids = lax.broadcasted_iota(jnp.int32, (1, B), 1)
        onehot = (batch_ref[...] == gids).astype(jnp.float32)

        pieces = [Eb]
        for hd in range(NH):
            pieces.append(M * Eb[:, hd:hd + 1])
        pieces.append(M)
        pieces.append(jnp.ones((_BLK, 1), jnp.float32))
        pieces.append(jnp.zeros((_BLK, _ACC_C - (NH + NH * H + H + 1)),
                                jnp.float32))
        Y = jnp.concatenate(pieces, axis=1)

        contrib = lax.dot_general(
            onehot, Y, (((0,), (0,)), ((), ())),
            preferred_element_type=jnp.float32,
        )

        @pl.when(i == 0)
        def _():
            acc_ref[...] = jnp.zeros_like(acc_ref)

        acc_ref[...] += contrib

        @pl.when(i == _GRID - 1)
        def _():
            acc = acc_ref[...]
            S = acc[:, 0:NH]
            inv = jnp.where(S > 0.0, 1.0 / S, 0.0)
            att = jnp.zeros((B, H), jnp.float32)
            for hd in range(NH):
                Q = acc[:, NH + hd * H:NH + (hd + 1) * H]
                att = att + Q * inv[:, hd:hd + 1]
            att = att * (1.0 / NH)
            sumM = acc[:, NH + NH * H:NH + NH * H + H]
            cnt = acc[:, NH + NH * H + H:NH + NH * H + H + 1]
            gm = sumM / jnp.maximum(cnt, 1.0)
            emb = jnp.concatenate([att, gm], axis=1)
            hc = jnp.maximum(
                jnp.dot(emb, c1w_ref[...], preferred_element_type=jnp.float32)
                + c1b_ref[...], 0.0)
            hc = jnp.maximum(
                jnp.dot(hc, c2w_ref[...], preferred_element_type=jnp.float32)
                + c2b_ref[...], 0.0)
            o_ref[...] = jnp.dot(
                hc, c3w_ref[...], preferred_element_type=jnp.float32
            ) + c3b_ref[...]

    return pl.pallas_call(
        body,
        grid=(_GRID,),
        in_specs=[
            _row_spec(H),
            _row_spec(H),
            _row_spec(H),
            _row_spec(1),
            _full_spec(1, H),
            _full_spec(1, H),
            _full_spec(1, H),
            _full_spec(H, H),
            _full_spec(1, H),
            _full_spec(H, NH),
            _full_spec(1, NH),
            _row_spec(1),
            _full_spec(2 * H, 256),
            _full_spec(1, 256),
            _full_spec(256, H),
            _full_spec(1, H),
            _full_spec(H, 2),
            _full_spec(1, 2),
        ],
        out_specs=_full_spec(B, 2),
        out_shape=jax.ShapeDtypeStruct((B, 2), jnp.float32),
        scratch_shapes=[pltpu.VMEM((B, _ACC_C), jnp.float32)],
    )(a, g, h_prev, dis, bias, ln_g, ln_b, W1, b1, W2, b2, batch2d,
      c1W, c1b, c2W, c2b, c3W, c3b)


def kernel(x, edge_index, edge_attr, batch, proj_W, proj_b, proj_g, proj_bb,
           gcn_W0, gcn_b0, ln_g0, ln_b0,
           gcn_W1, gcn_b1, ln_g1, ln_b1,
           gcn_W2, gcn_b2, ln_g2, ln_b2,
           W1, b1, W2, b2, c1W, c1b, c2W, c2b, c3W, c3b):
    row = edge_index[0]
    col = edge_index[1]
    ew = edge_attr[:, 0]
    rowr_d = row.reshape(NW, CHUNKS_D, K)
    colr_d = col.reshape(NW, CHUNKS_D, K)
    wr_d = ew.reshape(NW, CHUNKS_D, K)
    rowr_a = row.reshape(NS, CHUNKS_A, 1, K)
    colr_a = col.reshape(NS, CHUNKS_A, 1, K)
    wr_a = ew.reshape(NS, CHUNKS_A, 1, K)

    r1 = lambda v: v.reshape(1, -1)

    degp = _sc_deg(colr_d, wr_d)[:, :N]
    h0 = _tc_proj(x, proj_W, r1(proj_b), r1(proj_g), r1(proj_bb))
    g0, dis = _tc_gdis(h0, gcn_W0, degp[0].reshape(N, 1), degp[1].reshape(N, 1))

    a = _sc_agg(g0, rowr_a, colr_a, wr_a)[:N]
    h1, g1 = _tc_layer(a, g0, h0, dis, r1(gcn_b0), r1(ln_g0),
                       r1(ln_b0), gcn_W1, residual=False)
    a = _sc_agg(g1, rowr_a, colr_a, wr_a)[:N]
    h2, g2 = _tc_layer(a, g1, h1, dis, r1(gcn_b1), r1(ln_g1),
                       r1(ln_b1), gcn_W2, residual=True)
    a = _sc_agg(g2, rowr_a, colr_a, wr_a)[:N]
    out = _tc_final(a, g2, h2, dis, r1(gcn_b2), r1(ln_g2), r1(ln_b2),
                    W1, r1(b1), W2, r1(b2), batch.reshape(N, 1),
                    c1W, r1(c1b), c2W, r1(c2b), c3W, r1(c3b))
    return out


# speedup vs baseline: 1.6142x; 1.6142x over previous
import functools

import jax
import jax.numpy as jnp
from jax import lax
from jax.experimental import pallas as pl
from jax.experimental.pallas import tpu as pltpu
from jax.experimental.pallas import tpu_sc as plsc

N = 10000
E = 320000
D_IN = 1310
H = 128
B = 512
NH = 4

NC = 2
NS = 16
LANES = 16
NW = NC * NS
K = 80
CHUNKS_D = E // (NW * K)
CHUNKS_A = E // (NW * K)
NP = 10240
RPT = NP // NS


def _wid():
    return lax.axis_index("s") * NC + lax.axis_index("c")


def _sc_deg_body(colr, wr, out, colbuf, wbuf, zb, degsh, sem):
    cid = lax.axis_index("c")
    sid = lax.axis_index("s")
    wid = _wid()

    pltpu.sync_copy(colr.at[wid], colbuf)
    pltpu.sync_copy(wr.at[wid], wbuf)

    zeros16 = jnp.zeros((LANES,), jnp.float32)

    @pl.loop(0, RPT // LANES)
    def _(j):
        zb[pl.ds(j * LANES, LANES)] = zeros16

    pltpu.sync_copy(zb, degsh.at[pl.ds(sid * RPT, RPT)])

    plsc.subcore_barrier()

    for base in range(0, CHUNKS_D, 25):
        for k in range(base, base + 25):
            pltpu.make_async_copy(
                wbuf.at[k], degsh.at[colbuf.at[k]], sem
            ).start(add=True)
        for k in range(base, base + 25):
            pltpu.make_async_copy(wbuf.at[k], degsh.at[colbuf.at[k]], sem).wait()

    plsc.subcore_barrier()

    pltpu.sync_copy(
        degsh.at[pl.ds(sid * RPT, RPT)],
        out.at[cid].at[pl.ds(sid * RPT, RPT)],
    )


def _sc_agg_body(g, rowr, colr, wr, out, rb0, rb1, cb0, cb1, wb0, wb1,
                 G0, G1, Sbuf, aggsh, semg0, semg1, semi0, semi1):
    cid = lax.axis_index("c")
    sid = lax.axis_index("s")
    wid = _wid()

    zeros16 = jnp.zeros((LANES,), jnp.float32)
    lanes = lax.iota(jnp.int32, LANES)

    @pl.loop(0, K)
    def _(r):
        for c in range(H // LANES):
            G0[r, pl.ds(c * LANES, LANES)] = zeros16

    for rep in range(RPT // K):
        pltpu.sync_copy(G0, aggsh.at[pl.ds(sid * RPT + rep * K, K)])

    plsc.subcore_barrier()

    src_row = rowr.at[wid]
    src_col = colr.at[wid]
    src_w = wr.at[wid]

    def idx_start(k, rb, cb, wb, semi):
        pltpu.make_async_copy(src_row.at[k], rb, semi).start()
        pltpu.make_async_copy(src_col.at[k], cb, semi).start()
        pltpu.make_async_copy(src_w.at[k], wb, semi).start()

    def idx_wait(k, rb, cb, wb, semi):
        pltpu.make_async_copy(src_row.at[k], rb, semi).wait()
        pltpu.make_async_copy(src_col.at[k], cb, semi).wait()
        pltpu.make_async_copy(src_w.at[k], wb, semi).wait()

    def gather_start(rb, Gbuf, semg):
        pltpu.make_async_copy(g.at[rb.at[0]], Gbuf, semg).start()

    def gather_wait(rb, Gbuf, semg):
        pltpu.make_async_copy(g.at[rb.at[0]], Gbuf, semg).wait()

    zero16i = jnp.zeros((LANES,), jnp.int32)

    def scale_and_scatter(cb, wb, Gbuf):
        @plsc.parallel_loop(0, K, unroll=2)
        def _(e):
            e16 = jnp.full((LANES,), e, jnp.int32)
            we = plsc.load_gather(wb, [zero16i, e16])
            for c in range(H // LANES):
                Sbuf[e, pl.ds(c * LANES, LANES)] = (
                    Gbuf[e, pl.ds(c * LANES, LANES)] * we
                )

        pltpu.sync_copy(Sbuf, aggsh.at[cb.at[0]], add=True)

    bufs = ((rb0, cb0, wb0, G0, semg0, semi0),
            (rb1, cb1, wb1, G1, semg1, semi1))

    idx_start(0, rb0, cb0, wb0, semi0)
    idx_wait(0, rb0, cb0, wb0, semi0)
    gather_start(rb0, G0, semg0)
    idx_start(1, rb1, cb1, wb1, semi1)

    @pl.loop(0, CHUNKS_A, step=2)
    def _(k):
        for half in range(2):
            kk = k + half
            rb, cb, wb, Gbuf, semg, semi = bufs[half]
            rbn, cbn, wbn, Gn, semgn, semin = bufs[1 - half]

            @pl.when(kk + 1 < CHUNKS_A)
            def _():
                idx_wait(kk + 1, rbn, cbn, wbn, semin)
                gather_start(rbn, Gn, semgn)

            @pl.when(kk < CHUNKS_A)
            def _():
                gather_wait(rb, Gbuf, semg)
                scale_and_scatter(cb, wb, Gbuf)

            @pl.when(kk + 2 < CHUNKS_A)
            def _():
                idx_start(kk + 2, rb, cb, wb, semi)

    plsc.subcore_barrier()

    pltpu.sync_copy(
        aggsh.at[pl.ds(sid * RPT, RPT)],
        out.at[cid].at[pl.ds(sid * RPT, RPT)],
    )


@functools.cache
def _sc_mesh(num_cores):
    return plsc.VectorSubcoreMesh(
        core_axis_name="c", subcore_axis_name="s", num_cores=num_cores,
        num_subcores=NS,
    )


@functools.cache
def _sc_deg_kernel():
    return pl.kernel(
        _sc_deg_body,
        out_type=jax.ShapeDtypeStruct((NC, NP), jnp.float32),
        mesh=_sc_mesh(NC),
        scratch_types=[
            pltpu.VMEM((CHUNKS_D, K), jnp.int32),
            pltpu.VMEM((CHUNKS_D, K), jnp.float32),
            pltpu.VMEM((RPT,), jnp.float32),
            pltpu.VMEM_SHARED((NP,), jnp.float32),
            pltpu.SemaphoreType.DMA,
        ],
        compiler_params=pltpu.CompilerParams(needs_layout_passes=False),
    )


def _sc_deg(colr, wr):
    return _sc_deg_kernel()(colr, wr)


@functools.cache
def _sc_agg_kernel():
    return pl.kernel(
        _sc_agg_body,
        out_type=jax.ShapeDtypeStruct((NC, NP, H), jnp.float32),
        mesh=_sc_mesh(NC),
        scratch_types=[
            pltpu.VMEM((1, K), jnp.int32),
            pltpu.VMEM((1, K), jnp.int32),
            pltpu.VMEM((1, K), jnp.int32),
            pltpu.VMEM((1, K), jnp.int32),
            pltpu.VMEM((1, K), jnp.float32),
            pltpu.VMEM((1, K), jnp.float32),
            pltpu.VMEM((K, H), jnp.float32),
            pltpu.VMEM((K, H), jnp.float32),
            pltpu.VMEM((K, H), jnp.float32),
            pltpu.VMEM_SHARED((NP, H), jnp.float32),
            pltpu.SemaphoreType.DMA,
            pltpu.SemaphoreType.DMA,
            pltpu.SemaphoreType.DMA,
            pltpu.SemaphoreType.DMA,
        ],
        compiler_params=pltpu.CompilerParams(needs_layout_passes=False),
    )


def _sc_agg(g, rowr, colr, wr):
    return _sc_agg_kernel()(g, rowr, colr, wr)


_GRID = 10
_BLK = N // _GRID


def _ln(x, gamma, beta):
    mu = jnp.mean(x, axis=-1, keepdims=True)
    d = x - mu
    var = jnp.mean(d * d, axis=-1, keepdims=True)
    return d * lax.rsqrt(var + 1e-5) * gamma + beta


def _row_spec(cols):
    return pl.BlockSpec((_BLK, cols), lambda i: (i, 0))


def _full_spec(r, c):
    return pl.BlockSpec((r, c), lambda i: (0, 0))


def _tc_proj(x, proj_W, proj_b, proj_g, proj_bb):
    def body(x_ref, w_ref, b_ref, g_ref, bb_ref, o_ref):
        h = jnp.dot(x_ref[...], w_ref[...], preferred_element_type=jnp.float32)
        h = h + b_ref[...]
        o_ref[...] = jnp.maximum(_ln(h, g_ref[...], bb_ref[...]), 0.0)

    return pl.pallas_call(
        body,
        grid=(_GRID,),
        in_specs=[
            _row_spec(D_IN),
            _full_spec(D_IN, H),
            _full_spec(1, H),
            _full_spec(1, H),
            _full_spec(1, H),
        ],
        out_specs=_row_spec(H),
        out_shape=jax.ShapeDtypeStruct((N, H), jnp.float32),
    )(x, proj_W, proj_b, proj_g, proj_bb)


def _tc_gdis(h, W, deg0, deg1):
    def body(h_ref, w_ref, d0_ref, d1_ref, g_ref, dis_ref):
        deg = d0_ref[...] + d1_ref[...] + 1.0
        dis = lax.rsqrt(deg)
        dis_ref[...] = dis
        g_ref[...] = jnp.dot(
            h_ref[...], w_ref[...], preferred_element_type=jnp.float32
        ) * dis

    return pl.pallas_call(
        body,
        grid=(_GRID,),
        in_specs=[
            _row_spec(H),
            _full_spec(H, H),
            _row_spec(1),
            _row_spec(1),
        ],
        out_specs=[_row_spec(H), _row_spec(1)],
        out_shape=[
            jax.ShapeDtypeStruct((N, H), jnp.float32),
            jax.ShapeDtypeStruct((N, 1), jnp.float32),
        ],
    )(h, W, deg0, deg1)


def _tc_layer(a0, a1, g, h_prev, dis, bias, ln_g, ln_b, W_next, residual):
    def body(a0_ref, a1_ref, g_ref, hp_ref, dis_ref, b_ref, lg_ref, lb_ref,
             w_ref, h_ref, gn_ref):
        dis = dis_ref[...]
        xn = dis * (a0_ref[...] + a1_ref[...] + g_ref[...]) + b_ref[...]
        if residual:
            xn = xn + hp_ref[...]
        h = jnp.maximum(_ln(xn, lg_ref[...], lb_ref[...]), 0.0)
        h_ref[...] = h
        gn_ref[...] = jnp.dot(
            h, w_ref[...], preferred_element_type=jnp.float32
        ) * dis

    return pl.pallas_call(
        body,
        grid=(_GRID,),
        in_specs=[
            _row_spec(H),
            _row_spec(H),
            _row_spec(H),
            _row_spec(H),
            _row_spec(1),
            _full_spec(1, H),
            _full_spec(1, H),
            _full_spec(1, H),
            _full_spec(H, H),
        ],
        out_specs=[_row_spec(H), _row_spec(H)],
        out_shape=[
            jax.ShapeDtypeStruct((N, H), jnp.float32),
            jax.ShapeDtypeStruct((N, H), jnp.float32),
        ],
    )(a0, a1, g, h_prev, dis, bias, ln_g, ln_b, W_next)


_ACC_C = 768


def _tc_final(a0, a1, g, h_prev, dis, bias, ln_g, ln_b, W1, b1, W2, b2,
              batch2d, c1W, c1b, c2W, c2b, c3W, c3b):
    def body(a0_ref, a1_ref, g_ref, hp_ref, dis_ref, b_ref, lg_ref, lb_ref,
             w1_ref, b1_ref, w2_ref, b2_ref, batch_ref,
             c1w_ref, c1b_ref, c2w_ref, c2b_ref, c3w_ref, c3b_ref,
             o_ref, acc_ref):
        i = pl.program_id(0)
        xn = dis_ref[...] * (a0_ref[...] + a1_ref[...] + g_ref[...]) \
            + b_ref[...]
        xn = xn + hp_ref[...]
        M = jnp.maximum(_ln(xn, lg_ref[...], lb_ref[...]), 0.0)

        t = jnp.tanh(
            jnp.dot(M, w1_ref[...], preferred_element_type=jnp.float32)
            + b1_ref[...]
        )
        scores = jnp.dot(t, w2_ref[...], preferred_element_type=jnp.float32) \
            + b2_ref[...]
        Eb = jnp.exp(scores)

        gids = lax.broadcasted_iota(jnp.int32, (1, B), 1)
        onehot = (batch_ref[...] == gids).astype(jnp.float32)

        pieces = [Eb]
        for hd in range(NH):
            pieces.append(M * Eb[:, hd:hd + 1])
        pieces.append(M)
        pieces.append(jnp.ones((_BLK, 1), jnp.float32))
        pieces.append(jnp.zeros((_BLK, _ACC_C - (NH + NH * H + H + 1)),
                                jnp.float32))
        Y = jnp.concatenate(pieces, axis=1)

        contrib = lax.dot_general(
            onehot, Y, (((0,), (0,)), ((), ())),
            preferred_element_type=jnp.float32,
        )

        @pl.when(i == 0)
        def _():
            acc_ref[...] = jnp.zeros_like(acc_ref)

        acc_ref[...] += contrib

        @pl.when(i == _GRID - 1)
        def _():
            acc = acc_ref[...]
            S = acc[:, 0:NH]
            inv = jnp.where(S > 0.0, 1.0 / S, 0.0)
            att = jnp.zeros((B, H), jnp.float32)
            for hd in range(NH):
                Q = acc[:, NH + hd * H:NH + (hd + 1) * H]
                att = att + Q * inv[:, hd:hd + 1]
            att = att * (1.0 / NH)
            sumM = acc[:, NH + NH * H:NH + NH * H + H]
            cnt = acc[:, NH + NH * H + H:NH + NH * H + H + 1]
            gm = sumM / jnp.maximum(cnt, 1.0)
            emb = jnp.concatenate([att, gm], axis=1)
            hc = jnp.maximum(
                jnp.dot(emb, c1w_ref[...], preferred_element_type=jnp.float32)
                + c1b_ref[...], 0.0)
            hc = jnp.maximum(
                jnp.dot(hc, c2w_ref[...], preferred_element_type=jnp.float32)
                + c2b_ref[...], 0.0)
            o_ref[...] = jnp.dot(
                hc, c3w_ref[...], preferred_element_type=jnp.float32
            ) + c3b_ref[...]

    return pl.pallas_call(
        body,
        grid=(_GRID,),
        in_specs=[
            _row_spec(H),
            _row_spec(H),
            _row_spec(H),
            _row_spec(H),
            _row_spec(1),
            _full_spec(1, H),
            _full_spec(1, H),
            _full_spec(1, H),
            _full_spec(H, H),
            _full_spec(1, H),
            _full_spec(H, NH),
            _full_spec(1, NH),
            _row_spec(1),
            _full_spec(2 * H, 256),
            _full_spec(1, 256),
            _full_spec(256, H),
            _full_spec(1, H),
            _full_spec(H, 2),
            _full_spec(1, 2),
        ],
        out_specs=_full_spec(B, 2),
        out_shape=jax.ShapeDtypeStruct((B, 2), jnp.float32),
        scratch_shapes=[pltpu.VMEM((B, _ACC_C), jnp.float32)],
    )(a0, a1, g, h_prev, dis, bias, ln_g, ln_b, W1, b1, W2, b2, batch2d,
      c1W, c1b, c2W, c2b, c3W, c3b)


def kernel(x, edge_index, edge_attr, batch, proj_W, proj_b, proj_g, proj_bb,
           gcn_W0, gcn_b0, ln_g0, ln_b0,
           gcn_W1, gcn_b1, ln_g1, ln_b1,
           gcn_W2, gcn_b2, ln_g2, ln_b2,
           W1, b1, W2, b2, c1W, c1b, c2W, c2b, c3W, c3b):
    row = edge_index[0]
    col = edge_index[1]
    ew = edge_attr[:, 0]
    rowr_d = row.reshape(NW, CHUNKS_D, K)
    colr_d = col.reshape(NW, CHUNKS_D, K)
    wr_d = ew.reshape(NW, CHUNKS_D, K)
    rowr_a = row.reshape(NW, CHUNKS_A, 1, K)
    colr_a = col.reshape(NW, CHUNKS_A, 1, K)
    wr_a = ew.reshape(NW, CHUNKS_A, 1, K)

    r1 = lambda v: v.reshape(1, -1)

    degp = _sc_deg(colr_d, wr_d)[:, :N]
    h0 = _tc_proj(x, proj_W, r1(proj_b), r1(proj_g), r1(proj_bb))
    g0, dis = _tc_gdis(h0, gcn_W0, degp[0].reshape(N, 1), degp[1].reshape(N, 1))

    a = _sc_agg(g0, rowr_a, colr_a, wr_a)
    h1, g1 = _tc_layer(a[0, :N], a[1, :N], g0, h0, dis, r1(gcn_b0), r1(ln_g0),
                       r1(ln_b0), gcn_W1, residual=False)
    a = _sc_agg(g1, rowr_a, colr_a, wr_a)
    h2, g2 = _tc_layer(a[0, :N], a[1, :N], g1, h1, dis, r1(gcn_b1), r1(ln_g1),
                       r1(ln_b1), gcn_W2, residual=True)
    a = _sc_agg(g2, rowr_a, colr_a, wr_a)
    out = _tc_final(a[0, :N], a[1, :N], g2, h2, dis, r1(gcn_b2), r1(ln_g2),
                    r1(ln_b2), W1, r1(b1), W2, r1(b2), batch.reshape(N, 1),
                    c1W, r1(c1b), c2W, r1(c2b), c3W, r1(c3b))
    return out


# speedup vs baseline: 1.8874x; 1.1692x over previous
import functools

import jax
import jax.numpy as jnp
from jax import lax
from jax.experimental import pallas as pl
from jax.experimental.pallas import tpu as pltpu
from jax.experimental.pallas import tpu_sc as plsc

N = 10000
E = 320000
D_IN = 1310
H = 128
B = 512
NH = 4

NC = 2
NS = 16
LANES = 16
NW = NC * NS
K = 80
CHUNKS_D = E // (NW * K)
CHUNKS_A = E // (NW * K)
NP = 10240
RPT = NP // NS


def _wid():
    return lax.axis_index("s") * NC + lax.axis_index("c")


def _sc_deg_body(colr, wr, out, colbuf, wbuf, zb, degsh, sem):
    cid = lax.axis_index("c")
    sid = lax.axis_index("s")
    wid = _wid()

    pltpu.sync_copy(colr.at[wid], colbuf)
    pltpu.sync_copy(wr.at[wid], wbuf)

    zeros16 = jnp.zeros((LANES,), jnp.float32)

    @pl.loop(0, RPT // LANES)
    def _(j):
        zb[pl.ds(j * LANES, LANES)] = zeros16

    pltpu.sync_copy(zb, degsh.at[pl.ds(sid * RPT, RPT)])

    plsc.subcore_barrier()

    for base in range(0, CHUNKS_D, 25):
        for k in range(base, base + 25):
            pltpu.make_async_copy(
                wbuf.at[k], degsh.at[colbuf.at[k]], sem
            ).start(add=True)
        for k in range(base, base + 25):
            pltpu.make_async_copy(wbuf.at[k], degsh.at[colbuf.at[k]], sem).wait()

    plsc.subcore_barrier()

    pltpu.sync_copy(
        degsh.at[pl.ds(sid * RPT, RPT)],
        out.at[cid].at[pl.ds(sid * RPT, RPT)],
    )


def _sc_agg_body(g, rowr, colr, wr, out, rb0, rb1, cb0, cb1, wb0, wb1,
                 cbs0, cbs1, G0, G1, S0, S1, aggsh,
                 semg0, semg1, semi0, semi1, sems0, sems1):
    cid = lax.axis_index("c")
    sid = lax.axis_index("s")
    wid = _wid()

    zeros16 = jnp.zeros((LANES,), jnp.float32)
    lanes = lax.iota(jnp.int32, LANES)

    @pl.loop(0, K)
    def _(r):
        for c in range(H // LANES):
            G0[r, pl.ds(c * LANES, LANES)] = zeros16

    for rep in range(RPT // K):
        pltpu.sync_copy(G0, aggsh.at[pl.ds(sid * RPT + rep * K, K)])

    plsc.subcore_barrier()

    src_row = rowr.at[wid]
    src_col = colr.at[wid]
    src_w = wr.at[wid]

    def idx_start(k, rb, cb, wb, semi):
        pltpu.make_async_copy(src_row.at[k], rb, semi).start()
        pltpu.make_async_copy(src_col.at[k], cb, semi).start()
        pltpu.make_async_copy(src_w.at[k], wb, semi).start()

    def idx_wait(k, rb, cb, wb, semi):
        pltpu.make_async_copy(src_row.at[k], rb, semi).wait()
        pltpu.make_async_copy(src_col.at[k], cb, semi).wait()
        pltpu.make_async_copy(src_w.at[k], wb, semi).wait()

    def gather_start(rb, Gbuf, semg):
        pltpu.make_async_copy(g.at[rb.at[0]], Gbuf, semg).start()

    def gather_wait(rb, Gbuf, semg):
        pltpu.make_async_copy(g.at[rb.at[0]], Gbuf, semg).wait()

    zero16i = jnp.zeros((LANES,), jnp.int32)

    def scale(cb, wb, cbs, Gbuf, Sbuf):
        for c in range(K // LANES):
            cbs[0, pl.ds(c * LANES, LANES)] = cb[0, pl.ds(c * LANES, LANES)]

        @plsc.parallel_loop(0, K, unroll=2)
        def _(e):
            e16 = jnp.full((LANES,), e, jnp.int32)
            we = plsc.load_gather(wb, [zero16i, e16])
            for c in range(H // LANES):
                Sbuf[e, pl.ds(c * LANES, LANES)] = (
                    Gbuf[e, pl.ds(c * LANES, LANES)] * we
                )

    def scatter_start(cbs, Sbuf, sems):
        pltpu.make_async_copy(Sbuf, aggsh.at[cbs.at[0]], sems).start(add=True)

    def scatter_wait(cbs, Sbuf, sems):
        pltpu.make_async_copy(Sbuf, aggsh.at[cbs.at[0]], sems).wait()

    bufs = ((rb0, cb0, wb0, cbs0, G0, S0, semg0, semi0, sems0),
            (rb1, cb1, wb1, cbs1, G1, S1, semg1, semi1, sems1))

    idx_start(0, rb0, cb0, wb0, semi0)
    idx_wait(0, rb0, cb0, wb0, semi0)
    gather_start(rb0, G0, semg0)
    idx_start(1, rb1, cb1, wb1, semi1)

    @pl.loop(0, CHUNKS_A, step=2)
    def _(k):
        for half in range(2):
            kk = k + half
            rb, cb, wb, cbs, Gbuf, Sbuf, semg, semi, sems = bufs[half]
            rbn, cbn, wbn, _cbsn, Gn, _Sn, semgn, semin, _semsn = \
                bufs[1 - half]

            @pl.when(kk + 1 < CHUNKS_A)
            def _():
                idx_wait(kk + 1, rbn, cbn, wbn, semin)
                gather_start(rbn, Gn, semgn)

            @pl.when(kk < CHUNKS_A)
            def _():
                gather_wait(rb, Gbuf, semg)

                @pl.when(kk >= 2)
                def _():
                    scatter_wait(cbs, Sbuf, sems)

                scale(cb, wb, cbs, Gbuf, Sbuf)
                scatter_start(cbs, Sbuf, sems)

            @pl.when(kk + 2 < CHUNKS_A)
            def _():
                idx_start(kk + 2, rb, cb, wb, semi)

    scatter_wait(cbs0, S0, sems0)
    scatter_wait(cbs1, S1, sems1)

    plsc.subcore_barrier()

    pltpu.sync_copy(
        aggsh.at[pl.ds(sid * RPT, RPT)],
        out.at[cid].at[pl.ds(sid * RPT, RPT)],
    )


@functools.cache
def _sc_mesh(num_cores):
    return plsc.VectorSubcoreMesh(
        core_axis_name="c", subcore_axis_name="s", num_cores=num_cores,
        num_subcores=NS,
    )


@functools.cache
def _sc_deg_kernel():
    return pl.kernel(
        _sc_deg_body,
        out_type=jax.ShapeDtypeStruct((NC, NP), jnp.float32),
        mesh=_sc_mesh(NC),
        scratch_types=[
            pltpu.VMEM((CHUNKS_D, K), jnp.int32),
            pltpu.VMEM((CHUNKS_D, K), jnp.float32),
            pltpu.VMEM((RPT,), jnp.float32),
            pltpu.VMEM_SHARED((NP,), jnp.float32),
            pltpu.SemaphoreType.DMA,
        ],
        compiler_params=pltpu.CompilerParams(needs_layout_passes=False),
    )


def _sc_deg(colr, wr):
    return _sc_deg_kernel()(colr, wr)


@functools.cache
def _sc_agg_kernel():
    return pl.kernel(
        _sc_agg_body,
        out_type=jax.ShapeDtypeStruct((NC, NP, H), jnp.float32),
        mesh=_sc_mesh(NC),
        scratch_types=[
            pltpu.VMEM((1, K), jnp.int32),
            pltpu.VMEM((1, K), jnp.int32),
            pltpu.VMEM((1, K), jnp.int32),
            pltpu.VMEM((1, K), jnp.int32),
            pltpu.VMEM((1, K), jnp.float32),
            pltpu.VMEM((1, K), jnp.float32),
            pltpu.VMEM((1, K), jnp.int32),
            pltpu.VMEM((1, K), jnp.int32),
            pltpu.VMEM((K, H), jnp.float32),
            pltpu.VMEM((K, H), jnp.float32),
            pltpu.VMEM((K, H), jnp.float32),
            pltpu.VMEM((K, H), jnp.float32),
            pltpu.VMEM_SHARED((NP, H), jnp.float32),
            pltpu.SemaphoreType.DMA,
            pltpu.SemaphoreType.DMA,
            pltpu.SemaphoreType.DMA,
            pltpu.SemaphoreType.DMA,
            pltpu.SemaphoreType.DMA,
            pltpu.SemaphoreType.DMA,
        ],
        compiler_params=pltpu.CompilerParams(needs_layout_passes=False),
    )


def _sc_agg(g, rowr, colr, wr):
    return _sc_agg_kernel()(g, rowr, colr, wr)


_GRID = 10
_BLK = N // _GRID


def _ln(x, gamma, beta):
    mu = jnp.mean(x, axis=-1, keepdims=True)
    d = x - mu
    var = jnp.mean(d * d, axis=-1, keepdims=True)
    return d * lax.rsqrt(var + 1e-5) * gamma + beta


def _row_spec(cols):
    return pl.BlockSpec((_BLK, cols), lambda i: (i, 0))


def _full_spec(r, c):
    return pl.BlockSpec((r, c), lambda i: (0, 0))


def _tc_proj(x, proj_W, proj_b, proj_g, proj_bb):
    def body(x_ref, w_ref, b_ref, g_ref, bb_ref, o_ref):
        h = jnp.dot(x_ref[...], w_ref[...], preferred_element_type=jnp.float32)
        h = h + b_ref[...]
        o_ref[...] = jnp.maximum(_ln(h, g_ref[...], bb_ref[...]), 0.0)

    return pl.pallas_call(
        body,
        grid=(_GRID,),
        in_specs=[
            _row_spec(D_IN),
            _full_spec(D_IN, H),
            _full_spec(1, H),
            _full_spec(1, H),
            _full_spec(1, H),
        ],
        out_specs=_row_spec(H),
        out_shape=jax.ShapeDtypeStruct((N, H), jnp.float32),
    )(x, proj_W, proj_b, proj_g, proj_bb)


def _tc_gdis(h, W, deg0, deg1):
    def body(h_ref, w_ref, d0_ref, d1_ref, g_ref, dis_ref):
        deg = d0_ref[...] + d1_ref[...] + 1.0
        dis = lax.rsqrt(deg)
        dis_ref[...] = dis
        g_ref[...] = jnp.dot(
            h_ref[...], w_ref[...], preferred_element_type=jnp.float32
        ) * dis

    return pl.pallas_call(
        body,
        grid=(_GRID,),
        in_specs=[
            _row_spec(H),
            _full_spec(H, H),
            _row_spec(1),
            _row_spec(1),
        ],
        out_specs=[_row_spec(H), _row_spec(1)],
        out_shape=[
            jax.ShapeDtypeStruct((N, H), jnp.float32),
            jax.ShapeDtypeStruct((N, 1), jnp.float32),
        ],
    )(h, W, deg0, deg1)


def _tc_layer(a0, a1, g, h_prev, dis, bias, ln_g, ln_b, W_next, residual):
    def body(a0_ref, a1_ref, g_ref, hp_ref, dis_ref, b_ref, lg_ref, lb_ref,
             w_ref, h_ref, gn_ref):
        dis = dis_ref[...]
        xn = dis * (a0_ref[...] + a1_ref[...] + g_ref[...]) + b_ref[...]
        if residual:
            xn = xn + hp_ref[...]
        h = jnp.maximum(_ln(xn, lg_ref[...], lb_ref[...]), 0.0)
        h_ref[...] = h
        gn_ref[...] = jnp.dot(
            h, w_ref[...], preferred_element_type=jnp.float32
        ) * dis

    return pl.pallas_call(
        body,
        grid=(_GRID,),
        in_specs=[
            _row_spec(H),
            _row_spec(H),
            _row_spec(H),
            _row_spec(H),
            _row_spec(1),
            _full_spec(1, H),
            _full_spec(1, H),
            _full_spec(1, H),
            _full_spec(H, H),
        ],
        out_specs=[_row_spec(H), _row_spec(H)],
        out_shape=[
            jax.ShapeDtypeStruct((N, H), jnp.float32),
            jax.ShapeDtypeStruct((N, H), jnp.float32),
        ],
    )(a0, a1, g, h_prev, dis, bias, ln_g, ln_b, W_next)


_ACC_C = 768


def _tc_final(a0, a1, g, h_prev, dis, bias, ln_g, ln_b, W1, b1, W2, b2,
              batch2d, c1W, c1b, c2W, c2b, c3W, c3b):
    def body(a0_ref, a1_ref, g_ref, hp_ref, dis_ref, b_ref, lg_ref, lb_ref,
             w1_ref, b1_ref, w2_ref, b2_ref, batch_ref,
             c1w_ref, c1b_ref, c2w_ref, c2b_ref, c3w_ref, c3b_ref,
             o_ref, acc_ref):
        i = pl.program_id(0)
        xn = dis_ref[...] * (a0_ref[...] + a1_ref[...] + g_ref[...]) \
            + b_ref[...]
        xn = xn + hp_ref[...]
        M = jnp.maximum(_ln(xn, lg_ref[...], lb_ref[...]), 0.0)

        t = jnp.tanh(
            jnp.dot(M, w1_ref[...], preferred_element_type=jnp.float32)
            + b1_ref[...]
        )
        scores = jnp.dot(t, w2_ref[...], preferred_element_type=jnp.float32) \
            + b2_ref[...]
        Eb = jnp.exp(scores)

        gids = lax.broadcasted_iota(jnp.int32, (1, B), 1)
        onehot = (batch_ref[...] == gids).astype(jnp.float32)

        pieces = [Eb]
        for hd in range(NH):
            pieces.append(M * Eb[:, hd:hd + 1])
        pieces.append(M)
        pieces.append(jnp.ones((_BLK, 1), jnp.float32))
        pieces.append(jnp.zeros((_BLK, _ACC_C - (NH + NH * H + H + 1)),
                                jnp.float32))
        Y = jnp.concatenate(pieces, axis=1)

        contrib = lax.dot_general(
            onehot, Y, (((0,), (0,)), ((), ())),
            preferred_element_type=jnp.float32,
        )

        @pl.when(i == 0)
        def _():
            acc_ref[...] = jnp.zeros_like(acc_ref)

        acc_ref[...] += contrib

        @pl.when(i == _GRID - 1)
        def _():
            acc = acc_ref[...]
            S = acc[:, 0:NH]
            inv = jnp.where(S > 0.0, 1.0 / S, 0.0)
            att = jnp.zeros((B, H), jnp.float32)
            for hd in range(NH):
                Q = acc[:, NH + hd * H:NH + (hd + 1) * H]
                att = att + Q * inv[:, hd:hd + 1]
            att = att * (1.0 / NH)
            sumM = acc[:, NH + NH * H:NH + NH * H + H]
            cnt = acc[:, NH + NH * H + H:NH + NH * H + H + 1]
            gm = sumM / jnp.maximum(cnt, 1.0)
            emb = jnp.concatenate([att, gm], axis=1)
            hc = jnp.maximum(
                jnp.dot(emb, c1w_ref[...], preferred_element_type=jnp.float32)
                + c1b_ref[...], 0.0)
            hc = jnp.maximum(
                jnp.dot(hc, c2w_ref[...], preferred_element_type=jnp.float32)
                + c2b_ref[...], 0.0)
            o_ref[...] = jnp.dot(
                hc, c3w_ref[...], preferred_element_type=jnp.float32
            ) + c3b_ref[...]

    return pl.pallas_call(
        body,
        grid=(_GRID,),
        in_specs=[
            _row_spec(H),
            _row_spec(H),
            _row_spec(H),
            _row_spec(H),
            _row_spec(1),
            _full_spec(1, H),
            _full_spec(1, H),
            _full_spec(1, H),
            _full_spec(H, H),
            _full_spec(1, H),
            _full_spec(H, NH),
            _full_spec(1, NH),
            _row_spec(1),
            _full_spec(2 * H, 256),
            _full_spec(1, 256),
            _full_spec(256, H),
            _full_spec(1, H),
            _full_spec(H, 2),
            _full_spec(1, 2),
        ],
        out_specs=_full_spec(B, 2),
        out_shape=jax.ShapeDtypeStruct((B, 2), jnp.float32),
        scratch_shapes=[pltpu.VMEM((B, _ACC_C), jnp.float32)],
    )(a0, a1, g, h_prev, dis, bias, ln_g, ln_b, W1, b1, W2, b2, batch2d,
      c1W, c1b, c2W, c2b, c3W, c3b)


def kernel(x, edge_index, edge_attr, batch, proj_W, proj_b, proj_g, proj_bb,
           gcn_W0, gcn_b0, ln_g0, ln_b0,
           gcn_W1, gcn_b1, ln_g1, ln_b1,
           gcn_W2, gcn_b2, ln_g2, ln_b2,
           W1, b1, W2, b2, c1W, c1b, c2W, c2b, c3W, c3b):
    row = edge_index[0]
    col = edge_index[1]
    ew = edge_attr[:, 0]
    rowr_d = row.reshape(NW, CHUNKS_D, K)
    colr_d = col.reshape(NW, CHUNKS_D, K)
    wr_d = ew.reshape(NW, CHUNKS_D, K)
    rowr_a = row.reshape(NW, CHUNKS_A, 1, K)
    colr_a = col.reshape(NW, CHUNKS_A, 1, K)
    wr_a = ew.reshape(NW, CHUNKS_A, 1, K)

    r1 = lambda v: v.reshape(1, -1)

    degp = _sc_deg(colr_d, wr_d)[:, :N]
    h0 = _tc_proj(x, proj_W, r1(proj_b), r1(proj_g), r1(proj_bb))
    g0, dis = _tc_gdis(h0, gcn_W0, degp[0].reshape(N, 1), degp[1].reshape(N, 1))

    a = _sc_agg(g0, rowr_a, colr_a, wr_a)
    h1, g1 = _tc_layer(a[0, :N], a[1, :N], g0, h0, dis, r1(gcn_b0), r1(ln_g0),
                       r1(ln_b0), gcn_W1, residual=False)
    a = _sc_agg(g1, rowr_a, colr_a, wr_a)
    h2, g2 = _tc_layer(a[0, :N], a[1, :N], g1, h1, dis, r1(gcn_b1), r1(ln_g1),
                       r1(ln_b1), gcn_W2, residual=True)
    a = _sc_agg(g2, rowr_a, colr_a, wr_a)
    out = _tc_final(a[0, :N], a[1, :N], g2, h2, dis, r1(gcn_b2), r1(ln_g2),
                    r1(ln_b2), W1, r1(b1), W2, r1(b2), batch.reshape(N, 1),
                    c1W, r1(c1b), c2W, r1(c2b), c3W, r1(c3b))
    return out
